# baseline (speedup 1.0000x reference)
import functools

import jax
import jax.numpy as jnp
from jax.experimental import pallas as pl
from jax.experimental.pallas import tpu as pltpu

_NUM_CLASSES = 81
_TOP_K = 200
_SLOTS = 256
_NMS_THRESH = 0.45
_CONF_THRESH = 0.01
_V0 = 0.1
_V1 = 0.2
_ROWS = 128
_COLS = 160
_NEG = -jnp.inf
_BIG = 1e9


def _to_col(x_row, ident):
    return jnp.sum(ident * x_row, axis=1, keepdims=True)


def _detect_body(scores_ref, loc_ref, prior_ref, out_ref,
                 s_ref, bx_ref, iou_ref):
    loc = loc_ref[0]
    pr = prior_ref[...]
    cx = pr[0] + loc[0] * _V0 * pr[2]
    cy = pr[1] + loc[1] * _V0 * pr[3]
    w = pr[2] * jnp.exp(loc[2] * _V1)
    h = pr[3] * jnp.exp(loc[3] * _V1)
    dx1 = cx - w * 0.5
    dy1 = cy - h * 0.5
    bx_ref[0] = dx1
    bx_ref[1] = dy1
    bx_ref[2] = dx1 + w
    bx_ref[3] = dy1 + h

    st = scores_ref[0, 0]
    st = jnp.where(st > _CONF_THRESH, st, _NEG)
    s_ref[...] = st

    iota_r = jax.lax.broadcasted_iota(jnp.int32, (1, _ROWS), 1).astype(jnp.float32)
    iota_c = jax.lax.broadcasted_iota(jnp.int32, (1, _COLS), 1).astype(jnp.float32)
    lane_s = jax.lax.broadcasted_iota(jnp.int32, (1, _SLOTS), 1).astype(jnp.float32)

    rm_col = jnp.max(st, axis=1, keepdims=True)
    ra_col = jnp.min(
        jnp.where(st == rm_col, iota_c, _BIG), axis=1, keepdims=True)
    identr = jnp.where(
        jax.lax.broadcasted_iota(jnp.int32, (_ROWS, _ROWS), 0)
        == jax.lax.broadcasted_iota(jnp.int32, (_ROWS, _ROWS), 1),
        1.0, 0.0)
    rowmax = jnp.max(jnp.where(identr > 0.0, rm_col, _NEG), axis=0,
                     keepdims=True)
    rowarg = jnp.min(jnp.where(identr > 0.0, ra_col, _BIG), axis=0,
                     keepdims=True)

    zrow = jnp.zeros((1, _SLOTS), jnp.float32)

    def sel_body(i, carry):
        rowmax, rowarg, v, x1, y1, x2, y2 = carry
        m = jnp.max(rowmax)
        combo = jnp.where(rowmax == m, rowarg * float(_ROWS) + iota_r, _BIG)
        pbest = jnp.min(combo)
        c = jnp.floor(pbest / float(_ROWS))
        l = pbest - float(_ROWS) * c
        li = l.astype(jnp.int32)
        lmask = iota_r == l
        cmask = iota_c == c
        slot = lane_s == jnp.float32(i)

        v = jnp.where(slot, m, v)
        x1 = jnp.where(slot, jnp.sum(jnp.where(cmask, bx_ref[0, pl.ds(li, 1), :], 0.0)), x1)
        y1 = jnp.where(slot, jnp.sum(jnp.where(cmask, bx_ref[1, pl.ds(li, 1), :], 0.0)), y1)
        x2 = jnp.where(slot, jnp.sum(jnp.where(cmask, bx_ref[2, pl.ds(li, 1), :], 0.0)), x2)
        y2 = jnp.where(slot, jnp.sum(jnp.where(cmask, bx_ref[3, pl.ds(li, 1), :], 0.0)), y2)

        srow = jnp.where(cmask, _NEG, s_ref[pl.ds(li, 1), :])
        s_ref[pl.ds(li, 1), :] = srow
        nm = jnp.max(srow)
        na = jnp.min(jnp.where(srow == nm, iota_c, _BIG))
        rowmax = jnp.where(lmask, nm, rowmax)
        rowarg = jnp.where(lmask, na, rowarg)
        return rowmax, rowarg, v, x1, y1, x2, y2

    init = (rowmax, rowarg, jnp.full((1, _SLOTS), _NEG, jnp.float32),
            zrow, zrow, zrow, zrow)
    _, _, v, x1, y1, x2, y2 = jax.lax.fori_loop(0, _TOP_K, sel_body, init)

    ident = jnp.where(
        jax.lax.broadcasted_iota(jnp.int32, (_SLOTS, _SLOTS), 0)
        == jax.lax.broadcasted_iota(jnp.int32, (_SLOTS, _SLOTS), 1),
        1.0, 0.0)
    x1c = _to_col(x1, ident)
    y1c = _to_col(y1, ident)
    x2c = _to_col(x2, ident)
    y2c = _to_col(y2, ident)

    area_r = jnp.maximum(x2 - x1, 0.0) * jnp.maximum(y2 - y1, 0.0)
    area_c = jnp.maximum(x2c - x1c, 0.0) * jnp.maximum(y2c - y1c, 0.0)
    iw = jnp.maximum(jnp.minimum(x2c, x2) - jnp.maximum(x1c, x1), 0.0)
    ih = jnp.maximum(jnp.minimum(y2c, y2) - jnp.maximum(y1c, y1), 0.0)
    inter = iw * ih
    union = jnp.maximum(area_c + area_r - inter, 1e-12)
    iou_ref[...] = inter / union

    valid_f = jnp.where(v > _CONF_THRESH, 1.0, 0.0)

    def nms_body(i, alive_f):
        row = iou_ref[pl.ds(i, 1), :]
        fi = jnp.float32(i)
        alive_i = jnp.sum(jnp.where(lane_s == fi, alive_f, 0.0))
        gate = jnp.where(alive_i > 0.0, 1.0, 0.0)
        supp = jnp.where((row > _NMS_THRESH) & (lane_s > fi), gate, 0.0)
        return alive_f * (1.0 - supp)

    af = jax.lax.fori_loop(0, _TOP_K, nms_body, valid_f)
    alive = af > 0.0

    tri = jnp.where(
        jax.lax.broadcasted_iota(jnp.int32, (_SLOTS, _SLOTS), 0)
        <= jax.lax.broadcasted_iota(jnp.int32, (_SLOTS, _SLOTS), 1),
        1.0, 0.0)
    pos = jnp.dot(af, tri, preferred_element_type=jnp.float32) - 1.0

    pos_c = _to_col(pos, ident)
    alive_c = _to_col(af, ident) > 0.0
    perm = (pos_c == lane_s) & alive_c
    pf = jnp.where(perm, 1.0, 0.0)

    def compact(e_row):
        e = jnp.where(alive, e_row, 0.0)
        return jnp.sum(pf * _to_col(e, ident), axis=0, keepdims=True)

    out = jnp.concatenate(
        [compact(v), compact(x1), compact(y1), compact(x2), compact(y2),
         jnp.zeros((3, _SLOTS), jnp.float32)],
        axis=0,
    )
    out_ref[0, 0] = out


@functools.partial(jax.jit, static_argnames=("interpret",))
def kernel(loc_data, conf_data, prior_data, interpret=False):
    b, n, _ = loc_data.shape
    ncls = conf_data.shape[-1]
    npad = _ROWS * _COLS

    conf = conf_data.reshape(b, n, ncls).transpose(0, 2, 1)[:, 1:]
    conf = jnp.pad(conf, ((0, 0), (0, 0), (0, npad - n)))
    scores = conf.reshape(b, ncls - 1, _COLS, _ROWS).transpose(0, 1, 3, 2)

    loc_t = jnp.pad(loc_data.transpose(0, 2, 1), ((0, 0), (0, 0), (0, npad - n)))
    loc_t = loc_t.reshape(b, 4, _COLS, _ROWS).transpose(0, 1, 3, 2)
    pr_t = jnp.pad(prior_data.transpose(1, 0), ((0, 0), (0, npad - n)))
    pr_t = pr_t.reshape(4, _COLS, _ROWS).transpose(0, 2, 1)

    res = pl.pallas_call(
        _detect_body,
        grid=(b, ncls - 1),
        in_specs=[
            pl.BlockSpec((1, 1, _ROWS, _COLS), lambda i, j: (i, j, 0, 0)),
            pl.BlockSpec((1, 4, _ROWS, _COLS), lambda i, j: (i, 0, 0, 0)),
            pl.BlockSpec((4, _ROWS, _COLS), lambda i, j: (0, 0, 0)),
        ],
        out_specs=pl.BlockSpec((1, 1, 8, _SLOTS), lambda i, j: (i, j, 0, 0)),
        out_shape=jax.ShapeDtypeStruct((b, ncls - 1, 8, _SLOTS), jnp.float32),
        scratch_shapes=[
            pltpu.VMEM((_ROWS, _COLS), jnp.float32),
            pltpu.VMEM((4, _ROWS, _COLS), jnp.float32),
            pltpu.VMEM((_SLOTS, _SLOTS), jnp.float32),
        ],
        compiler_params=pltpu.CompilerParams(
            dimension_semantics=("parallel", "parallel")),
        interpret=interpret,
    )(scores, loc_t, pr_t)

    cls_out = res.transpose(0, 1, 3, 2)[:, :, :_TOP_K, :5]
    bg = jnp.zeros((b, 1, _TOP_K, 5), jnp.float32)
    return jnp.concatenate([bg, cls_out], axis=1)


# speedup vs baseline: 1.1053x; 1.1053x over previous
import jax
import jax.numpy as jnp
from jax.experimental import pallas as pl
from jax.experimental.pallas import tpu as pltpu

_NUM_CLASSES = 81
_TOP_K = 200
_SLOTS = 256
_NMS_THRESH = 0.45
_CONF_THRESH = 0.01
_V0 = 0.1
_V1 = 0.2
_ROWS = 160
_LANES = 128
_NEG = -jnp.inf
_BIG = 1e9


def _to_col(x_row, ident):
    return jnp.sum(ident * x_row, axis=1, keepdims=True)


def _detect_body(scores_ref, loc_ref, prior_ref, out_ref, iou_ref):
    loc = loc_ref[0]
    pr = prior_ref[...]
    cx = pr[0] + loc[0] * _V0 * pr[2]
    cy = pr[1] + loc[1] * _V0 * pr[3]
    w = pr[2] * jnp.exp(loc[2] * _V1)
    h = pr[3] * jnp.exp(loc[3] * _V1)
    dx1 = cx - w * 0.5
    dy1 = cy - h * 0.5
    dx2 = dx1 + w
    dy2 = dy1 + h

    s0 = scores_ref[0, 0]
    s0 = jnp.where(s0 > _CONF_THRESH, s0, _NEG)

    idx2d = (
        jax.lax.broadcasted_iota(jnp.int32, (_ROWS, _LANES), 0) * _LANES
        + jax.lax.broadcasted_iota(jnp.int32, (_ROWS, _LANES), 1)
    ).astype(jnp.float32)
    lane_s = jax.lax.broadcasted_iota(
        jnp.int32, (1, _SLOTS), 1).astype(jnp.float32)

    zrow = jnp.zeros((1, _SLOTS), jnp.float32)

    def sel_body(i, carry):
        s, v, x1, y1, x2, y2 = carry
        m = jnp.max(s)
        j = jnp.min(jnp.where(s == m, idx2d, _BIG))
        onehot = idx2d == j
        f = jnp.where(onehot, 1.0, 0.0)
        slot = lane_s == jnp.float32(i)
        v = jnp.where(slot, m, v)
        x1 = jnp.where(slot, jnp.sum(f * dx1), x1)
        y1 = jnp.where(slot, jnp.sum(f * dy1), y1)
        x2 = jnp.where(slot, jnp.sum(f * dx2), x2)
        y2 = jnp.where(slot, jnp.sum(f * dy2), y2)
        s = jnp.where(onehot, _NEG, s)
        return s, v, x1, y1, x2, y2

    init = (s0, jnp.full((1, _SLOTS), _NEG, jnp.float32), zrow, zrow, zrow, zrow)
    _, v, x1, y1, x2, y2 = jax.lax.fori_loop(0, _TOP_K, sel_body, init)

    ident = jnp.where(
        jax.lax.broadcasted_iota(jnp.int32, (_SLOTS, _SLOTS), 0)
        == jax.lax.broadcasted_iota(jnp.int32, (_SLOTS, _SLOTS), 1),
        1.0, 0.0)
    x1c = _to_col(x1, ident)
    y1c = _to_col(y1, ident)
    x2c = _to_col(x2, ident)
    y2c = _to_col(y2, ident)

    area_r = jnp.maximum(x2 - x1, 0.0) * jnp.maximum(y2 - y1, 0.0)
    area_c = jnp.maximum(x2c - x1c, 0.0) * jnp.maximum(y2c - y1c, 0.0)
    iw = jnp.maximum(jnp.minimum(x2c, x2) - jnp.maximum(x1c, x1), 0.0)
    ih = jnp.maximum(jnp.minimum(y2c, y2) - jnp.maximum(y1c, y1), 0.0)
    inter = iw * ih
    union = jnp.maximum(area_c + area_r - inter, 1e-12)
    iou_ref[...] = inter / union

    valid_f = jnp.where(v > _CONF_THRESH, 1.0, 0.0)

    def nms_body(i, alive_f):
        row = iou_ref[pl.ds(i, 1), :]
        fi = jnp.float32(i)
        alive_i = jnp.sum(jnp.where(lane_s == fi, alive_f, 0.0))
        gate = jnp.where(alive_i > 0.0, 1.0, 0.0)
        supp = jnp.where((row > _NMS_THRESH) & (lane_s > fi), gate, 0.0)
        return alive_f * (1.0 - supp)

    af = jax.lax.fori_loop(0, _TOP_K, nms_body, valid_f)
    alive = af > 0.0

    tri = jnp.where(
        jax.lax.broadcasted_iota(jnp.int32, (_SLOTS, _SLOTS), 0)
        <= jax.lax.broadcasted_iota(jnp.int32, (_SLOTS, _SLOTS), 1),
        1.0, 0.0)
    pos = jnp.dot(af, tri, preferred_element_type=jnp.float32) - 1.0

    pos_c = _to_col(pos, ident)
    alive_c = _to_col(af, ident) > 0.0
    perm = (pos_c == lane_s) & alive_c
    pf = jnp.where(perm, 1.0, 0.0)

    def compact(e_row):
        e = jnp.where(alive, e_row, 0.0)
        return jnp.sum(pf * _to_col(e, ident), axis=0, keepdims=True)

    out = jnp.concatenate(
        [compact(v), compact(x1), compact(y1), compact(x2), compact(y2),
         jnp.zeros((3, _SLOTS), jnp.float32)],
        axis=0,
    )
    out_ref[0, 0] = out


@jax.jit
def kernel(loc_data, conf_data, prior_data):
    b, n, _ = loc_data.shape
    ncls = conf_data.shape[-1]
    npad = _ROWS * _LANES

    conf = conf_data.reshape(b, n, ncls).transpose(0, 2, 1)[:, 1:]
    conf = jnp.pad(conf, ((0, 0), (0, 0), (0, npad - n)))
    scores = conf.reshape(b, ncls - 1, _ROWS, _LANES)

    loc_t = jnp.pad(loc_data.transpose(0, 2, 1), ((0, 0), (0, 0), (0, npad - n)))
    loc_t = loc_t.reshape(b, 4, _ROWS, _LANES)
    pr_t = jnp.pad(prior_data.transpose(1, 0), ((0, 0), (0, npad - n)))
    pr_t = pr_t.reshape(4, _ROWS, _LANES)

    res = pl.pallas_call(
        _detect_body,
        grid=(b, ncls - 1),
        in_specs=[
            pl.BlockSpec((1, 1, _ROWS, _LANES), lambda i, j: (i, j, 0, 0)),
            pl.BlockSpec((1, 4, _ROWS, _LANES), lambda i, j: (i, 0, 0, 0)),
            pl.BlockSpec((4, _ROWS, _LANES), lambda i, j: (0, 0, 0)),
        ],
        out_specs=pl.BlockSpec((1, 1, 8, _SLOTS), lambda i, j: (i, j, 0, 0)),
        out_shape=jax.ShapeDtypeStruct((b, ncls - 1, 8, _SLOTS), jnp.float32),
        scratch_shapes=[pltpu.VMEM((_SLOTS, _SLOTS), jnp.float32)],
        compiler_params=pltpu.CompilerParams(
            dimension_semantics=("parallel", "parallel")),
    )(scores, loc_t, pr_t)

    cls_out = res.transpose(0, 1, 3, 2)[:, :, :_TOP_K, :5]
    bg = jnp.zeros((b, 1, _TOP_K, 5), jnp.float32)
    return jnp.concatenate([bg, cls_out], axis=1)


# speedup vs baseline: 1.5127x; 1.3686x over previous
import jax
import jax.numpy as jnp
from jax.experimental import pallas as pl
from jax.experimental.pallas import tpu as pltpu

_NUM_CLASSES = 81
_TOP_K = 200
_SLOTS = 256
_NMS_THRESH = 0.45
_CONF_THRESH = 0.01
_V0 = 0.1
_V1 = 0.2
_ROWS = 160
_LANES = 128
_NEG = -jnp.inf
_BIG = 1e9


def _to_col(x_row, ident):
    return jnp.sum(ident * x_row, axis=1, keepdims=True)


def _detect_body(scores_ref, loc_ref, prior_ref, out_ref, iou_ref):
    loc = loc_ref[0]
    pr = prior_ref[...]
    cx = pr[0] + loc[0] * _V0 * pr[2]
    cy = pr[1] + loc[1] * _V0 * pr[3]
    w = pr[2] * jnp.exp(loc[2] * _V1)
    h = pr[3] * jnp.exp(loc[3] * _V1)
    dx1 = cx - w * 0.5
    dy1 = cy - h * 0.5
    dx2 = dx1 + w
    dy2 = dy1 + h

    s0 = scores_ref[0, 0]
    s0 = jnp.where(s0 > _CONF_THRESH, s0, _NEG)

    idx2d = (
        jax.lax.broadcasted_iota(jnp.int32, (_ROWS, _LANES), 0) * _LANES
        + jax.lax.broadcasted_iota(jnp.int32, (_ROWS, _LANES), 1)
    ).astype(jnp.float32)
    lane_s = jax.lax.broadcasted_iota(
        jnp.int32, (1, _SLOTS), 1).astype(jnp.float32)

    zrow = jnp.zeros((1, _SLOTS), jnp.float32)

    def sel_body(i, carry):
        s, v, jr = carry
        m = jnp.max(s)
        j = jnp.min(jnp.where(s == m, idx2d, _BIG))
        slot = lane_s == jnp.float32(i)
        v = jnp.where(slot, m, v)
        jr = jnp.where(slot, j, jr)
        s = jnp.where(idx2d == j, _NEG, s)
        return s, v, jr

    init = (s0, jnp.full((1, _SLOTS), _NEG, jnp.float32), zrow)
    _, v, jr = jax.lax.fori_loop(0, _TOP_K, sel_body, init)

    ident = jnp.where(
        jax.lax.broadcasted_iota(jnp.int32, (_SLOTS, _SLOTS), 0)
        == jax.lax.broadcasted_iota(jnp.int32, (_SLOTS, _SLOTS), 1),
        1.0, 0.0)
    j_col = _to_col(jr, ident)
    r_col = jnp.floor(j_col / float(_LANES))
    l_col = j_col - float(_LANES) * r_col
    iota_row = jax.lax.broadcasted_iota(jnp.int32, (1, _ROWS), 1).astype(jnp.float32)
    iota_lane = jax.lax.broadcasted_iota(jnp.int32, (1, _LANES), 1).astype(jnp.float32)
    grow = jnp.where(r_col == iota_row, 1.0, 0.0)
    lmask = jnp.where(l_col == iota_lane, 1.0, 0.0)

    def gather(dec):
        rows = jnp.dot(grow, dec, preferred_element_type=jnp.float32,
                       precision=jax.lax.Precision.HIGHEST)
        return jnp.sum(rows * lmask, axis=1, keepdims=True)

    x1c = gather(dx1)
    y1c = gather(dy1)
    x2c = gather(dx2)
    y2c = gather(dy2)

    def to_row(x_col):
        return jnp.sum(ident * x_col, axis=0, keepdims=True)

    x1 = to_row(x1c)
    y1 = to_row(y1c)
    x2 = to_row(x2c)
    y2 = to_row(y2c)

    area_r = jnp.maximum(x2 - x1, 0.0) * jnp.maximum(y2 - y1, 0.0)
    area_c = jnp.maximum(x2c - x1c, 0.0) * jnp.maximum(y2c - y1c, 0.0)
    iw = jnp.maximum(jnp.minimum(x2c, x2) - jnp.maximum(x1c, x1), 0.0)
    ih = jnp.maximum(jnp.minimum(y2c, y2) - jnp.maximum(y1c, y1), 0.0)
    inter = iw * ih
    union = jnp.maximum(area_c + area_r - inter, 1e-12)
    iou_ref[...] = inter / union

    valid_f = jnp.where(v > _CONF_THRESH, 1.0, 0.0)

    def nms_body(i, alive_f):
        row = iou_ref[pl.ds(i, 1), :]
        fi = jnp.float32(i)
        alive_i = jnp.sum(jnp.where(lane_s == fi, alive_f, 0.0))
        gate = jnp.where(alive_i > 0.0, 1.0, 0.0)
        supp = jnp.where((row > _NMS_THRESH) & (lane_s > fi), gate, 0.0)
        return alive_f * (1.0 - supp)

    af = jax.lax.fori_loop(0, _TOP_K, nms_body, valid_f)
    alive = af > 0.0

    tri = jnp.where(
        jax.lax.broadcasted_iota(jnp.int32, (_SLOTS, _SLOTS), 0)
        <= jax.lax.broadcasted_iota(jnp.int32, (_SLOTS, _SLOTS), 1),
        1.0, 0.0)
    pos = jnp.dot(af, tri, preferred_element_type=jnp.float32) - 1.0

    pos_c = _to_col(pos, ident)
    alive_c = _to_col(af, ident) > 0.0
    perm = (pos_c == lane_s) & alive_c
    pf = jnp.where(perm, 1.0, 0.0)

    def compact(e_row):
        e = jnp.where(alive, e_row, 0.0)
        return jnp.sum(pf * _to_col(e, ident), axis=0, keepdims=True)

    out = jnp.concatenate(
        [compact(v), compact(x1), compact(y1), compact(x2), compact(y2),
         jnp.zeros((3, _SLOTS), jnp.float32)],
        axis=0,
    )
    out_ref[0, 0] = out


@jax.jit
def kernel(loc_data, conf_data, prior_data):
    b, n, _ = loc_data.shape
    ncls = conf_data.shape[-1]
    npad = _ROWS * _LANES

    conf = conf_data.reshape(b, n, ncls).transpose(0, 2, 1)[:, 1:]
    conf = jnp.pad(conf, ((0, 0), (0, 0), (0, npad - n)))
    scores = conf.reshape(b, ncls - 1, _ROWS, _LANES)

    loc_t = jnp.pad(loc_data.transpose(0, 2, 1), ((0, 0), (0, 0), (0, npad - n)))
    loc_t = loc_t.reshape(b, 4, _ROWS, _LANES)
    pr_t = jnp.pad(prior_data.transpose(1, 0), ((0, 0), (0, npad - n)))
    pr_t = pr_t.reshape(4, _ROWS, _LANES)

    res = pl.pallas_call(
        _detect_body,
        grid=(b, ncls - 1),
        in_specs=[
            pl.BlockSpec((1, 1, _ROWS, _LANES), lambda i, j: (i, j, 0, 0)),
            pl.BlockSpec((1, 4, _ROWS, _LANES), lambda i, j: (i, 0, 0, 0)),
            pl.BlockSpec((4, _ROWS, _LANES), lambda i, j: (0, 0, 0)),
        ],
        out_specs=pl.BlockSpec((1, 1, 8, _SLOTS), lambda i, j: (i, j, 0, 0)),
        out_shape=jax.ShapeDtypeStruct((b, ncls - 1, 8, _SLOTS), jnp.float32),
        scratch_shapes=[pltpu.VMEM((_SLOTS, _SLOTS), jnp.float32)],
        compiler_params=pltpu.CompilerParams(
            dimension_semantics=("parallel", "parallel")),
    )(scores, loc_t, pr_t)

    cls_out = res.transpose(0, 1, 3, 2)[:, :, :_TOP_K, :5]
    bg = jnp.zeros((b, 1, _TOP_K, 5), jnp.float32)
    return jnp.concatenate([bg, cls_out], axis=1)
